# Initial kernel scaffold; baseline (speedup 1.0000x reference)
import jax
import jax.numpy as jnp
from jax.experimental import pallas as pl


def kernel(logits, labels):
    raise NotImplementedError("write your pallas kernel here")


# speedup vs baseline: 8.9530x; 8.9530x over previous
import functools
import math

import jax
import jax.numpy as jnp
from jax import lax
from jax.experimental import pallas as pl
from jax.experimental.pallas import tpu as pltpu
from jax.experimental.pallas import tpu_sc as plsc

_THRESH = float(math.log(1.0 / 0.7))
_MIN_KEPT = 131072
_NBINS = 1024
_INV_W = (_NBINS - 1) / _THRESH
_NW = 32
_N = 4 * 512 * 512
_PER_W = _N // _NW
_VECS = _PER_W // 16
_ROWS = 128


def _ce_body(lg_ref, lb_ref, loss_ref):
    lab = lb_ref[0]
    m = lg_ref[0, 0]
    for c in range(1, 19):
        m = jnp.maximum(m, lg_ref[0, c])
    se = jnp.zeros_like(m)
    picked = jnp.zeros_like(m)
    for c in range(19):
        v = lg_ref[0, c]
        se = se + jnp.exp(v - m)
        picked = picked + jnp.where(lab == c, v, 0.0)
    loss_ref[0] = m + jnp.log(se) - picked


def _ce_loss(logits, labels):
    b, c, h, w = logits.shape
    return pl.pallas_call(
        _ce_body,
        grid=(b, h // _ROWS),
        in_specs=[
            pl.BlockSpec((1, c, _ROWS, w), lambda i, r: (i, 0, r, 0)),
            pl.BlockSpec((1, _ROWS, w), lambda i, r: (i, r, 0)),
        ],
        out_specs=pl.BlockSpec((1, _ROWS, w), lambda i, r: (i, r, 0)),
        out_shape=jax.ShapeDtypeStruct((b, h, w), jnp.float32),
    )(logits, labels)


def _sc_hist_body(loss_hbm, counts_hbm, sums_hbm, chunk_v, cpriv, spriv,
                  cred, sred):
    wid = lax.axis_index("s") * 2 + lax.axis_index("c")
    base = pl.multiple_of(wid * _PER_W, 8)
    pltpu.sync_copy(loss_hbm.at[pl.ds(base, _PER_W)], chunk_v)

    zero = jnp.zeros((16,), jnp.float32)
    ones = jnp.full((16,), 1.0, jnp.float32)
    lane_base = lax.iota(jnp.int32, 16) * _NBINS

    def zbody(i, carry):
        off = pl.multiple_of(i * 16, 16)
        cpriv[pl.ds(off, 16)] = zero
        spriv[pl.ds(off, 16)] = zero
        return carry

    lax.fori_loop(0, _NBINS, zbody, 0)

    def body(i, carry):
        off = pl.multiple_of(i * 16, 16)
        v = chunk_v[pl.ds(off, 16)]
        bf = jnp.minimum(v * _INV_W, float(_NBINS + 8))
        bi = jnp.clip(bf.astype(jnp.int32), 0, _NBINS - 1)
        addr = lane_base + bi
        plsc.addupdate_scatter(cpriv, [addr], ones)
        plsc.addupdate_scatter(spriv, [addr], v)
        return carry

    lax.fori_loop(0, _VECS, body, 0)

    def rbody(p, carry):
        off = pl.multiple_of(p * 16, 16)
        acc_c = zero
        acc_s = zero
        for j in range(16):
            acc_c = acc_c + cpriv[pl.ds(j * _NBINS + off, 16)]
            acc_s = acc_s + spriv[pl.ds(j * _NBINS + off, 16)]
        cred[pl.ds(off, 16)] = acc_c
        sred[pl.ds(off, 16)] = acc_s
        return carry

    lax.fori_loop(0, _NBINS // 16, rbody, 0)

    pltpu.sync_copy(cred, counts_hbm.at[wid])
    pltpu.sync_copy(sred, sums_hbm.at[wid])


@functools.cache
def _sc_hist():
    return pl.kernel(
        _sc_hist_body,
        mesh=plsc.VectorSubcoreMesh(core_axis_name="c", subcore_axis_name="s"),
        compiler_params=pltpu.CompilerParams(needs_layout_passes=False),
        out_type=[
            jax.ShapeDtypeStruct((_NW, _NBINS), jnp.float32),
            jax.ShapeDtypeStruct((_NW, _NBINS), jnp.float32),
        ],
        scratch_types=[
            pltpu.VMEM((_PER_W,), jnp.float32),
            pltpu.VMEM((_NBINS * 16,), jnp.float32),
            pltpu.VMEM((_NBINS * 16,), jnp.float32),
            pltpu.VMEM((_NBINS,), jnp.float32),
            pltpu.VMEM((_NBINS,), jnp.float32),
        ],
    )


def _combine_body(c_ref, s_ref, out_ref):
    kf = jnp.float32(_MIN_KEPT)
    c = jnp.sum(c_ref[...], axis=0, keepdims=True)
    s = jnp.sum(s_ref[...], axis=0, keepdims=True)
    ii = lax.broadcasted_iota(jnp.int32, (_NBINS, _NBINS), 0)
    jj = lax.broadcasted_iota(jnp.int32, (_NBINS, _NBINS), 1)
    cb = jnp.broadcast_to(c, (_NBINS, _NBINS))
    sb = jnp.broadcast_to(s, (_NBINS, _NBINS))
    sa = jnp.sum(jnp.where(jj >= ii, cb, 0.0), axis=1, keepdims=True)
    ss = jnp.sum(jnp.where(jj >= ii, sb, 0.0), axis=1, keepdims=True)
    c_col = jnp.sum(jnp.where(jj == ii, cb, 0.0), axis=1, keepdims=True)
    s_col = jnp.sum(jnp.where(jj == ii, sb, 0.0), axis=1, keepdims=True)
    i_col = lax.broadcasted_iota(jnp.int32, (_NBINS, 1), 0)
    cut = jnp.max(jnp.where(sa >= kf, i_col, -1))
    oneh = (i_col == cut).astype(jnp.float32)
    c_cut = jnp.sum(oneh * c_col)
    s_cut = jnp.sum(oneh * s_col)
    sa_cut = jnp.sum(oneh * sa)
    ss_cut = jnp.sum(oneh * ss)
    above_c = sa_cut - c_cut
    above_s = ss_cut - s_cut
    r = kf - above_c
    mean_cut = s_cut / jnp.maximum(c_cut, 1.0)
    mean_topk = (above_s + r * mean_cut) / kf
    hard_h = (i_col == (_NBINS - 1)).astype(jnp.float32)
    n_hard = jnp.sum(hard_h * c_col)
    sum_hard = jnp.sum(hard_h * s_col)
    mean_hard = sum_hard / jnp.maximum(n_hard, 1.0)
    res = jnp.where(n_hard >= kf, mean_hard, mean_topk)
    out_ref[...] = jnp.broadcast_to(res, (1, 1))


def _combine(counts, sums):
    return pl.pallas_call(
        _combine_body,
        out_shape=jax.ShapeDtypeStruct((1, 1), jnp.float32),
    )(counts, sums)


def kernel(logits, labels):
    labels = labels.astype(jnp.int32)
    loss = _ce_loss(logits, labels)
    counts, sums = _sc_hist()(loss.reshape(-1))
    return _combine(counts, sums)[0, 0]


# speedup vs baseline: 17.0973x; 1.9097x over previous
import functools
import math

import jax
import jax.numpy as jnp
from jax import lax
from jax.experimental import pallas as pl
from jax.experimental.pallas import tpu as pltpu
from jax.experimental.pallas import tpu_sc as plsc

_THRESH = float(math.log(1.0 / 0.7))
_MIN_KEPT = 131072
_NBINS = 1024
_INV_W = (_NBINS - 1) / _THRESH
_NW = 32
_N = 4 * 512 * 512
_PER_W = _N // _NW
_VECS = _PER_W // 16
_ROWS = 128


def _ce_body(lg_ref, lb_ref, loss_ref):
    lab = lb_ref[0]
    m = lg_ref[0, 0]
    for c in range(1, 19):
        m = jnp.maximum(m, lg_ref[0, c])
    se = jnp.zeros_like(m)
    picked = jnp.zeros_like(m)
    for c in range(19):
        v = lg_ref[0, c]
        se = se + jnp.exp(v - m)
        picked = picked + jnp.where(lab == c, v, 0.0)
    loss_ref[0] = m + jnp.log(se) - picked


def _ce_loss(logits, labels):
    b, c, h, w = logits.shape
    return pl.pallas_call(
        _ce_body,
        grid=(b, h // _ROWS),
        in_specs=[
            pl.BlockSpec((1, c, _ROWS, w), lambda i, r: (i, 0, r, 0)),
            pl.BlockSpec((1, _ROWS, w), lambda i, r: (i, r, 0)),
        ],
        out_specs=pl.BlockSpec((1, _ROWS, w), lambda i, r: (i, r, 0)),
        out_shape=jax.ShapeDtypeStruct((b, h, w), jnp.float32),
    )(logits, labels)


def _sc_hist_body(loss_hbm, counts_hbm, sums_hbm, chunk_v, cpriv, spriv,
                  cred, sred):
    wid = lax.axis_index("s") * 2 + lax.axis_index("c")
    base = pl.multiple_of(wid * _PER_W, 8)
    pltpu.sync_copy(loss_hbm.at[pl.ds(base, _PER_W)], chunk_v)

    zero = jnp.zeros((16,), jnp.float32)
    ones = jnp.full((16,), 1.0, jnp.float32)
    lane = lax.iota(jnp.int32, 16)

    @plsc.parallel_loop(0, _NBINS, unroll=8)
    def _(i):
        off = pl.multiple_of(i * 16, 16)
        cpriv[pl.ds(off, 16)] = zero
        spriv[pl.ds(off, 16)] = zero

    @plsc.parallel_loop(0, _VECS, unroll=8)
    def _(i):
        off = pl.multiple_of(i * 16, 16)
        v = chunk_v[pl.ds(off, 16)]
        bf = jnp.minimum(v * _INV_W, float(_NBINS + 8))
        bi = jnp.clip(bf.astype(jnp.int32), 0, _NBINS - 1)
        addr = bi * 16 + lane
        plsc.addupdate_scatter(cpriv, [addr], ones)
        plsc.addupdate_scatter(spriv, [addr], v)

    diags = [lane * 16 + (lane ^ j) for j in range(16)]

    @plsc.parallel_loop(0, _NBINS // 16, unroll=2)
    def _(p):
        off = pl.multiple_of(p * 16, 16)
        base = off * 16
        acc_c = zero
        acc_s = zero
        for j in range(16):
            idx = base + diags[j]
            acc_c = acc_c + plsc.load_gather(cpriv, [idx])
            acc_s = acc_s + plsc.load_gather(spriv, [idx])
        cred[pl.ds(off, 16)] = acc_c
        sred[pl.ds(off, 16)] = acc_s

    pltpu.sync_copy(cred, counts_hbm.at[wid])
    pltpu.sync_copy(sred, sums_hbm.at[wid])


@functools.cache
def _sc_hist():
    return pl.kernel(
        _sc_hist_body,
        mesh=plsc.VectorSubcoreMesh(core_axis_name="c", subcore_axis_name="s"),
        compiler_params=pltpu.CompilerParams(needs_layout_passes=False),
        out_type=[
            jax.ShapeDtypeStruct((_NW, _NBINS), jnp.float32),
            jax.ShapeDtypeStruct((_NW, _NBINS), jnp.float32),
        ],
        scratch_types=[
            pltpu.VMEM((_PER_W,), jnp.float32),
            pltpu.VMEM((_NBINS * 16,), jnp.float32),
            pltpu.VMEM((_NBINS * 16,), jnp.float32),
            pltpu.VMEM((_NBINS,), jnp.float32),
            pltpu.VMEM((_NBINS,), jnp.float32),
        ],
    )


def _combine_body(c_ref, s_ref, out_ref):
    kf = jnp.float32(_MIN_KEPT)
    c = jnp.sum(c_ref[...], axis=0, keepdims=True)
    s = jnp.sum(s_ref[...], axis=0, keepdims=True)
    ii = lax.broadcasted_iota(jnp.int32, (_NBINS, _NBINS), 0)
    jj = lax.broadcasted_iota(jnp.int32, (_NBINS, _NBINS), 1)
    cb = jnp.broadcast_to(c, (_NBINS, _NBINS))
    sb = jnp.broadcast_to(s, (_NBINS, _NBINS))
    sa = jnp.sum(jnp.where(jj >= ii, cb, 0.0), axis=1, keepdims=True)
    ss = jnp.sum(jnp.where(jj >= ii, sb, 0.0), axis=1, keepdims=True)
    c_col = jnp.sum(jnp.where(jj == ii, cb, 0.0), axis=1, keepdims=True)
    s_col = jnp.sum(jnp.where(jj == ii, sb, 0.0), axis=1, keepdims=True)
    i_col = lax.broadcasted_iota(jnp.int32, (_NBINS, 1), 0)
    cut = jnp.max(jnp.where(sa >= kf, i_col, -1))
    oneh = (i_col == cut).astype(jnp.float32)
    c_cut = jnp.sum(oneh * c_col)
    s_cut = jnp.sum(oneh * s_col)
    sa_cut = jnp.sum(oneh * sa)
    ss_cut = jnp.sum(oneh * ss)
    above_c = sa_cut - c_cut
    above_s = ss_cut - s_cut
    r = kf - above_c
    mean_cut = s_cut / jnp.maximum(c_cut, 1.0)
    mean_topk = (above_s + r * mean_cut) / kf
    hard_h = (i_col == (_NBINS - 1)).astype(jnp.float32)
    n_hard = jnp.sum(hard_h * c_col)
    sum_hard = jnp.sum(hard_h * s_col)
    mean_hard = sum_hard / jnp.maximum(n_hard, 1.0)
    res = jnp.where(n_hard >= kf, mean_hard, mean_topk)
    out_ref[...] = jnp.broadcast_to(res, (1, 1))


def _combine(counts, sums):
    return pl.pallas_call(
        _combine_body,
        out_shape=jax.ShapeDtypeStruct((1, 1), jnp.float32),
    )(counts, sums)


def kernel(logits, labels):
    labels = labels.astype(jnp.int32)
    loss = _ce_loss(logits, labels)
    counts, sums = _sc_hist()(loss.reshape(-1))
    return _combine(counts, sums)[0, 0]


# speedup vs baseline: 17.8532x; 1.0442x over previous
import functools
import math

import jax
import jax.numpy as jnp
from jax import lax
from jax.experimental import pallas as pl
from jax.experimental.pallas import tpu as pltpu
from jax.experimental.pallas import tpu_sc as plsc

_THRESH = float(math.log(1.0 / 0.7))
_MIN_KEPT = 131072
_NBINS = 1024
_INV_W = (_NBINS - 1) / _THRESH
_NW = 32
_N = 4 * 512 * 512
_PER_W = _N // _NW
_VECS = _PER_W // 16
_ROWS = 256


def _ce_body(lg_ref, lb_ref, loss_ref):
    lab = lb_ref[0]
    m = lg_ref[0, 0]
    for c in range(1, 19):
        m = jnp.maximum(m, lg_ref[0, c])
    se = jnp.zeros_like(m)
    picked = jnp.zeros_like(m)
    for c in range(19):
        v = lg_ref[0, c]
        se = se + jnp.exp(v - m)
        picked = picked + jnp.where(lab == c, v, 0.0)
    loss_ref[0] = m + jnp.log(se) - picked


def _ce_loss(logits, labels):
    b, c, h, w = logits.shape
    return pl.pallas_call(
        _ce_body,
        grid=(b, h // _ROWS),
        in_specs=[
            pl.BlockSpec((1, c, _ROWS, w), lambda i, r: (i, 0, r, 0)),
            pl.BlockSpec((1, _ROWS, w), lambda i, r: (i, r, 0)),
        ],
        out_specs=pl.BlockSpec((1, _ROWS, w), lambda i, r: (i, r, 0)),
        out_shape=jax.ShapeDtypeStruct((b, h, w), jnp.float32),
    )(logits, labels)


def _sc_hist_body(loss_hbm, counts_hbm, sums_hbm, chunk_v, cpriv, spriv,
                  cred, sred):
    wid = lax.axis_index("s") * 2 + lax.axis_index("c")
    base = pl.multiple_of(wid * _PER_W, 8)
    pltpu.sync_copy(loss_hbm.at[pl.ds(base, _PER_W)], chunk_v)

    zero = jnp.zeros((16,), jnp.float32)
    ones = jnp.full((16,), 1.0, jnp.float32)
    lane = lax.iota(jnp.int32, 16)

    @plsc.parallel_loop(0, _NBINS, unroll=8)
    def _(i):
        off = pl.multiple_of(i * 16, 16)
        cpriv[pl.ds(off, 16)] = zero
        spriv[pl.ds(off, 16)] = zero

    @plsc.parallel_loop(0, _VECS, unroll=8)
    def _(i):
        off = pl.multiple_of(i * 16, 16)
        v = chunk_v[pl.ds(off, 16)]
        bf = jnp.minimum(v * _INV_W, float(_NBINS + 8))
        bi = jnp.clip(bf.astype(jnp.int32), 0, _NBINS - 1)
        addr = bi * 16 + lane
        plsc.addupdate_scatter(cpriv, [addr], ones)
        plsc.addupdate_scatter(spriv, [addr], v)

    diags = [lane * 16 + (lane ^ j) for j in range(16)]

    @plsc.parallel_loop(0, _NBINS // 16, unroll=2)
    def _(p):
        off = pl.multiple_of(p * 16, 16)
        base = off * 16
        acc_c = zero
        acc_s = zero
        for j in range(16):
            idx = base + diags[j]
            acc_c = acc_c + plsc.load_gather(cpriv, [idx])
            acc_s = acc_s + plsc.load_gather(spriv, [idx])
        cred[pl.ds(off, 16)] = acc_c
        sred[pl.ds(off, 16)] = acc_s

    pltpu.sync_copy(cred, counts_hbm.at[wid])
    pltpu.sync_copy(sred, sums_hbm.at[wid])


@functools.cache
def _sc_hist():
    return pl.kernel(
        _sc_hist_body,
        mesh=plsc.VectorSubcoreMesh(core_axis_name="c", subcore_axis_name="s"),
        compiler_params=pltpu.CompilerParams(needs_layout_passes=False),
        out_type=[
            jax.ShapeDtypeStruct((_NW, _NBINS), jnp.float32),
            jax.ShapeDtypeStruct((_NW, _NBINS), jnp.float32),
        ],
        scratch_types=[
            pltpu.VMEM((_PER_W,), jnp.float32),
            pltpu.VMEM((_NBINS * 16,), jnp.float32),
            pltpu.VMEM((_NBINS * 16,), jnp.float32),
            pltpu.VMEM((_NBINS,), jnp.float32),
            pltpu.VMEM((_NBINS,), jnp.float32),
        ],
    )


def _combine_body(c_ref, s_ref, out_ref):
    kf = jnp.float32(_MIN_KEPT)
    c = jnp.sum(c_ref[...], axis=0, keepdims=True)
    s = jnp.sum(s_ref[...], axis=0, keepdims=True)
    ii = lax.broadcasted_iota(jnp.int32, (_NBINS, _NBINS), 0)
    jj = lax.broadcasted_iota(jnp.int32, (_NBINS, _NBINS), 1)
    cb = jnp.broadcast_to(c, (_NBINS, _NBINS))
    sb = jnp.broadcast_to(s, (_NBINS, _NBINS))
    sa = jnp.sum(jnp.where(jj >= ii, cb, 0.0), axis=1, keepdims=True)
    ss = jnp.sum(jnp.where(jj >= ii, sb, 0.0), axis=1, keepdims=True)
    c_col = jnp.sum(jnp.where(jj == ii, cb, 0.0), axis=1, keepdims=True)
    s_col = jnp.sum(jnp.where(jj == ii, sb, 0.0), axis=1, keepdims=True)
    i_col = lax.broadcasted_iota(jnp.int32, (_NBINS, 1), 0)
    cut = jnp.max(jnp.where(sa >= kf, i_col, -1))
    oneh = (i_col == cut).astype(jnp.float32)
    c_cut = jnp.sum(oneh * c_col)
    s_cut = jnp.sum(oneh * s_col)
    sa_cut = jnp.sum(oneh * sa)
    ss_cut = jnp.sum(oneh * ss)
    above_c = sa_cut - c_cut
    above_s = ss_cut - s_cut
    r = kf - above_c
    mean_cut = s_cut / jnp.maximum(c_cut, 1.0)
    mean_topk = (above_s + r * mean_cut) / kf
    hard_h = (i_col == (_NBINS - 1)).astype(jnp.float32)
    n_hard = jnp.sum(hard_h * c_col)
    sum_hard = jnp.sum(hard_h * s_col)
    mean_hard = sum_hard / jnp.maximum(n_hard, 1.0)
    res = jnp.where(n_hard >= kf, mean_hard, mean_topk)
    out_ref[...] = jnp.broadcast_to(res, (1, 1))


def _combine(counts, sums):
    return pl.pallas_call(
        _combine_body,
        out_shape=jax.ShapeDtypeStruct((1, 1), jnp.float32),
    )(counts, sums)


def kernel(logits, labels):
    labels = labels.astype(jnp.int32)
    loss = _ce_loss(logits, labels)
    counts, sums = _sc_hist()(loss.reshape(-1))
    return _combine(counts, sums)[0, 0]
